# Initial kernel scaffold; baseline (speedup 1.0000x reference)
import jax
import jax.numpy as jnp
from jax.experimental import pallas as pl


def kernel(x, W, bias):
    raise NotImplementedError("write your pallas kernel here")


# speedup vs baseline: 1.2789x; 1.2789x over previous
import functools

import jax
import jax.numpy as jnp
from jax import lax
from jax.experimental import pallas as pl
from jax.experimental.pallas import tpu as pltpu
from jax.experimental.pallas import tpu_sc as plsc

BATCH = 16384
NUM_FIELDS = 26
FIELD_DIM = 40000
TOTAL = BATCH * NUM_FIELDS
NUM_WORKERS = 32
PER_TILE = TOTAL // NUM_WORKERS
ROWS_PER_TILE = BATCH // NUM_WORKERS
LANES = 16
NCHUNK = PER_TILE // LANES


def _tile_body(x_hbm, w_hbm, b_hbm, out_hbm, xv, gv, ov, bv, sem):
    wid = lax.axis_index("s") * 2 + lax.axis_index("c")
    base = wid * PER_TILE

    pltpu.sync_copy(x_hbm.at[pl.ds(base, PER_TILE)], xv)
    pltpu.sync_copy(b_hbm, bv.at[pl.ds(0, 1)])

    iota = lax.iota(jnp.int32, LANES)

    @pl.loop(0, NCHUNK)
    def _(j):
        o = j * LANES
        f = lax.rem(o + iota, NUM_FIELDS)
        xv[pl.ds(o, LANES)] = xv[pl.ds(o, LANES)] + f * FIELD_DIM

    pltpu.async_copy(w_hbm.at[xv], gv, sem).wait()

    bias = bv[pl.ds(0, LANES)][0]

    @pl.loop(0, ROWS_PER_TILE // LANES)
    def _(c):
        b0 = c * (LANES * NUM_FIELDS)
        acc = jnp.full((LANES,), bias, jnp.float32)
        for f in range(NUM_FIELDS):
            acc = acc + plsc.load_gather(gv, [iota * NUM_FIELDS + (b0 + f)])
        ov[pl.ds(c * LANES, LANES)] = acc

    pltpu.sync_copy(ov, out_hbm.at[pl.ds(wid * ROWS_PER_TILE, ROWS_PER_TILE)])


@jax.jit
def kernel(x, W, bias):
    x_flat = x.reshape(-1)
    w_flat = W.reshape(-1)
    fn = pl.kernel(
        _tile_body,
        out_type=jax.ShapeDtypeStruct((BATCH,), jnp.float32),
        mesh=plsc.VectorSubcoreMesh(core_axis_name="c", subcore_axis_name="s"),
        scratch_types=[
            pltpu.VMEM((PER_TILE,), jnp.int32),
            pltpu.VMEM((PER_TILE,), jnp.float32),
            pltpu.VMEM((ROWS_PER_TILE,), jnp.float32),
            pltpu.VMEM((LANES,), jnp.float32),
            pltpu.SemaphoreType.DMA,
        ],
        compiler_params=pltpu.CompilerParams(needs_layout_passes=False),
    )
    return fn(x_flat, w_flat, bias)


# speedup vs baseline: 1.5024x; 1.1748x over previous
import jax
import jax.numpy as jnp
from jax import lax
from jax.experimental import pallas as pl
from jax.experimental.pallas import tpu as pltpu
from jax.experimental.pallas import tpu_sc as plsc

BATCH = 16384
NUM_FIELDS = 26
FIELD_DIM = 40000
TOTAL_ROWS = NUM_FIELDS * FIELD_DIM
NUM_WORKERS = 32
PER_TILE = NUM_FIELDS * BATCH // NUM_WORKERS
ROWS_PER_TILE = BATCH // NUM_WORKERS
LANES = 16
CVEC = ROWS_PER_TILE // LANES


def _tile_body(x_hbm, w_hbm, b_hbm, out_hbm, xv, idxv, gv, ov, bv, sem):
    wid = lax.axis_index("s") * 2 + lax.axis_index("c")
    col0 = wid * ROWS_PER_TILE

    pltpu.sync_copy(x_hbm.at[:, pl.ds(col0, ROWS_PER_TILE)], xv)
    pltpu.sync_copy(b_hbm, bv.at[pl.ds(0, 1)])

    @pl.loop(0, NUM_FIELDS * CVEC)
    def _(j):
        f = j // CVEC
        o = (j - f * CVEC) * LANES
        idxv[pl.ds(f * ROWS_PER_TILE + o, LANES)] = (
            xv[f, pl.ds(o, LANES)] + f * FIELD_DIM
        )

    pltpu.async_copy(w_hbm.at[idxv], gv, sem).wait()

    bias = bv[pl.ds(0, LANES)][0]

    @pl.loop(0, CVEC)
    def _(c):
        o = c * LANES
        acc = jnp.full((LANES,), bias, jnp.float32)
        for f in range(NUM_FIELDS):
            acc = acc + gv[pl.ds(f * ROWS_PER_TILE + o, LANES)]
        ov[pl.ds(o, LANES)] = acc

    pltpu.sync_copy(ov, out_hbm.at[pl.ds(col0, ROWS_PER_TILE)])


@jax.jit
def kernel(x, W, bias):
    fn = pl.kernel(
        _tile_body,
        out_type=jax.ShapeDtypeStruct((BATCH,), jnp.float32),
        mesh=plsc.VectorSubcoreMesh(core_axis_name="c", subcore_axis_name="s"),
        scratch_types=[
            pltpu.VMEM((NUM_FIELDS, ROWS_PER_TILE), jnp.int32),
            pltpu.VMEM((PER_TILE,), jnp.int32),
            pltpu.VMEM((PER_TILE,), jnp.float32),
            pltpu.VMEM((ROWS_PER_TILE,), jnp.float32),
            pltpu.VMEM((LANES,), jnp.float32),
            pltpu.SemaphoreType.DMA,
        ],
        compiler_params=pltpu.CompilerParams(needs_layout_passes=False),
    )
    return fn(x.T, W.reshape(-1), bias)


# speedup vs baseline: 1.5874x; 1.0566x over previous
import functools

import jax
import jax.numpy as jnp
from jax import lax
from jax.experimental import pallas as pl
from jax.experimental.pallas import tpu as pltpu
from jax.experimental.pallas import tpu_sc as plsc

BATCH = 16384
NUM_FIELDS = 26
FIELD_DIM = 40000
F_HALF = NUM_FIELDS // 2
HALF_ROWS = F_HALF * FIELD_DIM
NUM_WORKERS = 32
ROWS_PER_TILE = BATCH // NUM_WORKERS
PER_TILE = F_HALF * ROWS_PER_TILE
LANES = 16
CVEC = ROWS_PER_TILE // LANES


def _make_body(field0):
    def body(x_hbm, w_hbm, p_hbm, out_hbm, xv, idxv, gv, pv, ov, sem):
        wid = lax.axis_index("s") * 2 + lax.axis_index("c")
        col0 = wid * ROWS_PER_TILE

        pltpu.sync_copy(x_hbm.at[:, pl.ds(col0, ROWS_PER_TILE)], xv)
        if field0 == 0:
            pltpu.sync_copy(p_hbm, pv.at[pl.ds(0, 1)])
        else:
            pltpu.sync_copy(p_hbm.at[pl.ds(col0, ROWS_PER_TILE)], pv)

        @pl.loop(0, F_HALF * CVEC)
        def _(j):
            f = j // CVEC
            o = (j - f * CVEC) * LANES
            idxv[pl.ds(f * ROWS_PER_TILE + o, LANES)] = (
                xv[field0 + f, pl.ds(o, LANES)] + f * FIELD_DIM
            )

        pltpu.async_copy(w_hbm.at[idxv], gv, sem).wait()

        if field0 == 0:
            bias = pv[pl.ds(0, LANES)][0]

        @pl.loop(0, CVEC)
        def _(c):
            o = c * LANES
            if field0 == 0:
                acc = jnp.full((LANES,), bias, jnp.float32)
            else:
                acc = pv[pl.ds(o, LANES)]
            for f in range(F_HALF):
                acc = acc + gv[pl.ds(f * ROWS_PER_TILE + o, LANES)]
            ov[pl.ds(o, LANES)] = acc

        pltpu.sync_copy(ov, out_hbm.at[pl.ds(col0, ROWS_PER_TILE)])

    return body


def _make_call(field0):
    psize = LANES if field0 == 0 else ROWS_PER_TILE
    return pl.kernel(
        _make_body(field0),
        out_type=jax.ShapeDtypeStruct((BATCH,), jnp.float32),
        mesh=plsc.VectorSubcoreMesh(core_axis_name="c", subcore_axis_name="s"),
        scratch_types=[
            pltpu.VMEM((NUM_FIELDS, ROWS_PER_TILE), jnp.int32),
            pltpu.VMEM((PER_TILE,), jnp.int32),
            pltpu.VMEM((PER_TILE,), jnp.float32),
            pltpu.VMEM((psize,), jnp.float32),
            pltpu.VMEM((ROWS_PER_TILE,), jnp.float32),
            pltpu.SemaphoreType.DMA,
        ],
        compiler_params=pltpu.CompilerParams(needs_layout_passes=False),
    )


@jax.jit
def kernel(x, W, bias):
    x_t = x.T
    w_lo = W[:HALF_ROWS].reshape(-1)
    (w2,) = lax.optimization_barrier((W,))
    w_hi = w2[HALF_ROWS:].reshape(-1)
    out_lo = _make_call(0)(x_t, w_lo, bias)
    return _make_call(F_HALF)(x_t, w_hi, out_lo)
